# baseline (speedup 1.0000x reference)
import dataclasses
import functools

import jax
import jax.numpy as jnp
import numpy as np
from jax import lax
from jax.experimental import pallas as pl
from jax.experimental.pallas import tpu as pltpu
from jax.experimental.pallas import tpu_sc as plsc

B, S, V, D = 4096, 200, 1000000, 64
NC, NS = 2, 16
NW = NC * NS
BB = 128
NBLK = S * (B // BB)
BLK_PER_W = NBLK // NW
BPS = B // BB
LANES = 16
NSLOT = 4
NIDX = NSLOT
WPAD = 133


def _positional_encoding() -> np.ndarray:
    pos = np.arange(S, dtype=np.float32)[:, None]
    i = np.arange(D, dtype=np.float32)[None, :]
    angle_rates = 1.0 / np.power(10000.0, (2.0 * np.floor(i / 2.0)) / np.float32(D))
    angle_rads = pos * angle_rates
    pe = np.zeros((S, D), dtype=np.float32)
    pe[:, 0::2] = np.sin(angle_rads[:, 0::2])
    pe[:, 1::2] = np.cos(angle_rads[:, 1::2])
    return pe


_PE = _positional_encoding()


def _sc_compiler_params():
    cp = pltpu.CompilerParams(use_tc_tiling_on_sc=False)
    if "needs_layout_passes" in pltpu.CompilerParams.__dataclass_fields__:
        cp = dataclasses.replace(cp, needs_layout_passes=False)
    return cp


def kernel(inputs, table):
    idx4 = (inputs.T.reshape(S // 8, 8, B // BB, BB)
            .transpose(0, 2, 1, 3))
    pe = jnp.asarray(_PE)

    mesh = plsc.VectorSubcoreMesh(core_axis_name="c", subcore_axis_name="s")

    @functools.partial(
        pl.kernel,
        out_type=jax.ShapeDtypeStruct((S, D // 8, B // BB, 8, BB), jnp.float32),
        mesh=mesh,
        compiler_params=_sc_compiler_params(),
        scratch_types=[
            pltpu.VMEM((S, D), jnp.float32),
        ]
        + [pltpu.VMEM((BB,), jnp.int32) for _ in range(NIDX)]
        + [pltpu.VMEM((BB, D), jnp.float32) for _ in range(NSLOT)]
        + [pltpu.VMEM((D // 8, 8, WPAD), jnp.float32) for _ in range(NSLOT)]
        + [pltpu.SemaphoreType.DMA for _ in range(NIDX + 2 * NSLOT)],
    )
    def run(idx_hbm, table_hbm, pe_hbm, out_hbm, pe_v, *bufs):
        o = 0
        ibuf = bufs[o:o + NIDX]; o += NIDX
        rows = bufs[o:o + NSLOT]; o += NSLOT
        wblk = bufs[o:o + NSLOT]; o += NSLOT
        isem = bufs[o:o + NIDX]; o += NIDX
        gsem = bufs[o:o + NSLOT]; o += NSLOT
        wsem = bufs[o:o + NSLOT]

        wid = lax.axis_index("s") * NC + lax.axis_index("c")
        gbase = wid * BLK_PER_W
        pltpu.sync_copy(pe_hbm, pe_v)

        lane = jnp.arange(LANES, dtype=jnp.int32)
        din_idx = lane % 8
        dt_base = lane // 8

        def tile_coords(m):
            h = gbase + m
            ts = h // 256
            tb = lax.rem(h // 8, BPS)
            s_in = lax.rem(h, 8)
            return ts, tb, s_in

        def idxload(m, q):
            ts, tb, s_in = tile_coords(m)
            return pltpu.make_async_copy(
                idx_hbm.at[ts, tb, s_in], ibuf[q], isem[q])

        def gather(m, p):
            return pltpu.make_async_copy(
                table_hbm.at[ibuf[p]], rows[p], gsem[p])

        def wb(m, p):
            ts, tb, s_in = tile_coords(m)
            s = 8 * ts + s_in
            return pltpu.make_async_copy(
                wblk[p].at[:, :, pl.ds(0, BB)], out_hbm.at[s, :, tb], wsem[p])

        def compute(m, p):
            ts, _, s_in = tile_coords(m)
            s = 8 * ts + s_in
            pe_vecs = [pe_v[s, pl.ds(g * LANES, LANES)] for g in range(D // LANES)]
            dt_vecs = [dt_base + 2 * g for g in range(D // LANES)]

            @pl.loop(0, BB, step=8)
            def _tok(t0):
                for tt in range(8):
                    t = t0 + tt
                    t_splat = jnp.full((LANES,), 0, dtype=jnp.int32) + t
                    for g in range(D // LANES):
                        v = rows[p][t, pl.ds(g * LANES, LANES)] + pe_vecs[g]
                        plsc.store_scatter(
                            wblk[p], [dt_vecs[g], din_idx, t_splat], v)

        def visit(m, p, first_round):
            gather(m, p).wait()
            idxload(m + NSLOT, p).start()
            if not first_round:
                wb(m - NSLOT, p).wait()
            compute(m, p)
            wb(m, p).start()
            idxload(m + NSLOT, p).wait()
            gather(m + NSLOT, p).start()

        for q in range(NIDX):
            idxload(q, q).start()
        for k in range(NSLOT):
            idxload(k, k).wait()
            gather(k, k).start()
        for k in range(NSLOT):
            visit(k, k, first_round=True)

        @pl.loop(NSLOT, BLK_PER_W - NSLOT, step=NSLOT)
        def _body(j):
            for k in range(NSLOT):
                visit(j + k, k, first_round=False)

        for k in range(NSLOT):
            m = BLK_PER_W - NSLOT + k
            gather(m, k).wait()
            wb(m - NSLOT, k).wait()
            compute(m, k)
            wb(m, k).start()
        for k in range(NSLOT):
            wb(BLK_PER_W - NSLOT + k, k).wait()

    out5d = run(idx4, table, pe)
    return out5d.transpose(2, 4, 0, 1, 3).reshape(B, S, D)
